# Initial kernel scaffold; baseline (speedup 1.0000x reference)
import jax
import jax.numpy as jnp
from jax.experimental import pallas as pl


def kernel(inputs, embeddings):
    raise NotImplementedError("write your pallas kernel here")


# speedup vs baseline: 2.4234x; 2.4234x over previous
import functools

import numpy as np
import jax
import jax.numpy as jnp
from jax import lax
from jax.experimental import pallas as pl
from jax.experimental.pallas import tpu as pltpu
from jax.experimental.pallas import tpu_sc as plsc

_LENGTH = 100
_DIM = 100
_PAD = 104
_BATCH = 4096
_ROWS = _BATCH * _LENGTH
_NW = 32
_RPW = _ROWS // _NW
_GCH = 128
_NGCH = _RPW // _GCH
_PREP = 4
_NPCH = _RPW // (_PREP * _LENGTH)


def _pe_full():
    pe = np.zeros((_LENGTH, _DIM))
    for pos in range(_LENGTH):
        for i in range(_DIM):
            pe[pos, i] = pos / np.power(10000, (i - i % 2) / _DIM)
    pe[:, 0::2] = np.sin(pe[:, 0::2])
    pe[:, 1::2] = np.cos(pe[:, 1::2])
    return pe.astype(np.float32)


def _pe_tail():
    return jnp.asarray(np.tile(_pe_full()[:, 4:], (_PREP, 1)))


def _pe_head():
    pe = _pe_full()
    tab = np.zeros((_LENGTH, 16), np.float32)
    for m in range(_LENGTH):
        for k in range(4):
            tab[m, 4 * k : 4 * k + 4] = pe[(m + k) % _LENGTH, :4]
    return jnp.asarray(tab)


def _sc_lookup(idx, table, pe_tail, pe_head):
    mesh = plsc.VectorSubcoreMesh(core_axis_name="c", subcore_axis_name="s")

    @functools.partial(
        pl.kernel,
        mesh=mesh,
        out_type=jax.ShapeDtypeStruct((_ROWS, 2 * _DIM), jnp.float32),
        scratch_types=[
            pltpu.VMEM((_NGCH, _GCH), jnp.int32),
            pltpu.VMEM((_GCH, _PAD), jnp.float32),
            pltpu.VMEM((_PREP * _LENGTH, 2 * _DIM - _PAD), jnp.float32),
            pltpu.VMEM((_LENGTH, 16), jnp.float32),
            pltpu.SemaphoreType.DMA,
        ],
        compiler_params=pltpu.CompilerParams(
            use_tc_tiling_on_sc=False, needs_layout_passes=False
        ),
    )
    def body(idx_hbm, table_hbm, pet_hbm, peh_hbm, out_hbm,
             idx_v, gbuf, petv, pehv, sem):
        wid = lax.axis_index("s") * 2 + lax.axis_index("c")
        base = wid * _RPW
        pltpu.sync_copy(idx_hbm.at[pl.ds(wid * _NGCH, _NGCH), :], idx_v)
        pltpu.sync_copy(pet_hbm, petv)
        pltpu.sync_copy(peh_hbm, pehv)

        lanes = lax.iota(jnp.int32, 16)
        row_off = lanes >> 2
        col_idx = _DIM + (lanes & 3)

        def gstep(g, carry):
            r0 = base + g * _GCH
            pltpu.async_copy(table_hbm.at[idx_v.at[g]], gbuf, sem).wait()
            m0 = lax.rem(g * _GCH, _LENGTH)

            def patch(q, carry2):
                m = lax.rem(m0 + 4 * q, _LENGTH)
                plsc.store_scatter(
                    gbuf, [4 * q + row_off, col_idx], pehv[m]
                )
                return carry2

            lax.fori_loop(0, _GCH // 4, patch, 0)
            pltpu.sync_copy(gbuf, out_hbm.at[pl.ds(r0, _GCH), pl.ds(0, _PAD)])
            return carry

        lax.fori_loop(0, _NGCH, gstep, 0)

        def pstep(i, carry):
            r0 = base + i * (_PREP * _LENGTH)
            pltpu.sync_copy(
                petv,
                out_hbm.at[pl.ds(r0, _PREP * _LENGTH), pl.ds(_PAD, 2 * _DIM - _PAD)],
            )
            return carry

        lax.fori_loop(0, _NPCH, pstep, 0)

    return body(idx, table, pe_tail, pe_head)


def kernel(inputs, embeddings):
    idx = inputs.reshape(_ROWS // _GCH, _GCH)
    tpad = jnp.pad(embeddings, ((0, 0), (0, _PAD - _DIM)))
    out = _sc_lookup(idx, tpad, _pe_tail(), _pe_head())
    return out.reshape(_BATCH, _LENGTH, 2 * _DIM)


# speedup vs baseline: 2.6042x; 1.0746x over previous
import functools

import numpy as np
import jax
import jax.numpy as jnp
from jax import lax
from jax.experimental import pallas as pl
from jax.experimental.pallas import tpu as pltpu
from jax.experimental.pallas import tpu_sc as plsc

_LENGTH = 100
_DIM = 100
_PAD = 104
_TAIL = 2 * _DIM - _PAD
_BATCH = 4096
_ROWS = _BATCH * _LENGTH
_NW = 32
_RPW = _ROWS // _NW
_GCH = 128
_NGCH = _RPW // _GCH
_NBUF = 4


def _pe_full():
    pe = np.zeros((_LENGTH, _DIM))
    for pos in range(_LENGTH):
        for i in range(_DIM):
            pe[pos, i] = pos / np.power(10000, (i - i % 2) / _DIM)
    pe[:, 0::2] = np.sin(pe[:, 0::2])
    pe[:, 1::2] = np.cos(pe[:, 1::2])
    return pe.astype(np.float32)


def _pe_tail():
    return jnp.asarray(np.tile(_pe_full()[:, 4:], (3, 1)))


def _pe_head():
    pe = _pe_full()
    tab = np.zeros((_LENGTH, 16), np.float32)
    for m in range(_LENGTH):
        for k in range(4):
            tab[m, 4 * k : 4 * k + 4] = pe[(m + k) % _LENGTH, :4]
    return jnp.asarray(tab)


def _sc_lookup(idx, table, pe_tail, pe_head):
    mesh = plsc.VectorSubcoreMesh(core_axis_name="c", subcore_axis_name="s")

    @functools.partial(
        pl.kernel,
        mesh=mesh,
        out_type=jax.ShapeDtypeStruct((_ROWS, 2 * _DIM), jnp.float32),
        scratch_types=[
            pltpu.VMEM((_NGCH, _GCH), jnp.int32),
            pltpu.VMEM((_NBUF, _GCH, _PAD), jnp.float32),
            pltpu.VMEM((3 * _LENGTH, _TAIL), jnp.float32),
            pltpu.VMEM((_LENGTH, 16), jnp.float32),
            pltpu.SemaphoreType.DMA((_NBUF,)),
            pltpu.SemaphoreType.DMA((_NBUF,)),
            pltpu.SemaphoreType.DMA,
        ],
        compiler_params=pltpu.CompilerParams(
            use_tc_tiling_on_sc=False, needs_layout_passes=False
        ),
    )
    def body(idx_hbm, table_hbm, pet_hbm, peh_hbm, out_hbm,
             idx_v, gbuf, petv, pehv, sg, sw, sp):
        wid = lax.axis_index("s") * 2 + lax.axis_index("c")
        base = wid * _RPW
        pltpu.sync_copy(idx_hbm.at[pl.ds(wid * _NGCH, _NGCH), :], idx_v)
        pltpu.sync_copy(pet_hbm, petv)
        pltpu.sync_copy(peh_hbm, pehv)

        lanes = lax.iota(jnp.int32, 16)
        row_off = lanes >> 2
        col_idx = _DIM + (lanes & 3)

        def gather_into(g, b):
            pltpu.async_copy(table_hbm.at[idx_v.at[g]], gbuf.at[b], sg.at[b])

        for k in range(_NBUF - 1):
            gather_into(k, k)

        def step(g, carry):
            b = lax.rem(g, _NBUF)
            bn = lax.rem(g + _NBUF - 1, _NBUF)
            m0 = lax.rem(g * _GCH, _LENGTH)
            r0 = base + g * _GCH

            @pl.when(g + _NBUF - 1 < _NGCH)
            def _fire():
                @pl.when(g >= 1)
                def _drain():
                    pltpu.make_async_copy(
                        gbuf.at[bn],
                        out_hbm.at[pl.ds(base + (g - 1) * _GCH, _GCH),
                                   pl.ds(0, _PAD)],
                        sw.at[bn],
                    ).wait()

                gather_into(g + _NBUF - 1, bn)

            pltpu.async_copy(
                petv.at[pl.ds(m0, _GCH), :],
                out_hbm.at[pl.ds(r0, _GCH), pl.ds(_PAD, _TAIL)],
                sp,
            )

            pltpu.make_async_copy(
                table_hbm.at[idx_v.at[g]], gbuf.at[b], sg.at[b]
            ).wait()

            def patch(q, carry2):
                m = lax.rem(m0 + 4 * q, _LENGTH)
                plsc.store_scatter(
                    gbuf.at[b], [4 * q + row_off, col_idx], pehv[m]
                )
                return carry2

            lax.fori_loop(0, _GCH // 4, patch, 0)

            pltpu.async_copy(
                gbuf.at[b],
                out_hbm.at[pl.ds(r0, _GCH), pl.ds(0, _PAD)],
                sw.at[b],
            )
            return carry

        lax.fori_loop(0, _NGCH, step, 0)

        for k in range(_NBUF):
            g = _NGCH - _NBUF + k
            pltpu.make_async_copy(
                gbuf.at[g % _NBUF],
                out_hbm.at[pl.ds(base + g * _GCH, _GCH), pl.ds(0, _PAD)],
                sw.at[g % _NBUF],
            ).wait()

        def pdrain(g, carry):
            m0 = lax.rem(g * _GCH, _LENGTH)
            pltpu.make_async_copy(
                petv.at[pl.ds(m0, _GCH), :],
                out_hbm.at[pl.ds(base + g * _GCH, _GCH), pl.ds(_PAD, _TAIL)],
                sp,
            ).wait()
            return carry

        lax.fori_loop(0, _NGCH, pdrain, 0)

    return body(idx, table, pe_tail, pe_head)


def kernel(inputs, embeddings):
    idx = inputs.reshape(_ROWS // _GCH, _GCH)
    tpad = jnp.pad(embeddings, ((0, 0), (0, _PAD - _DIM)))
    out = _sc_lookup(idx, tpad, _pe_tail(), _pe_head())
    return out.reshape(_BATCH, _LENGTH, 2 * _DIM)


# speedup vs baseline: 2.6082x; 1.0015x over previous
import functools

import numpy as np
import jax
import jax.numpy as jnp
from jax import lax
from jax.experimental import pallas as pl
from jax.experimental.pallas import tpu as pltpu
from jax.experimental.pallas import tpu_sc as plsc

_LENGTH = 100
_DIM = 100
_PAD = 104
_TAIL = 2 * _DIM - _PAD
_BATCH = 4096
_NW = 32
_SPW = _BATCH // _NW
_GS = 2
_GROWS = _GS * _LENGTH
_NG = _SPW // _GS
_NBUF = 4


def _pe_full():
    pe = np.zeros((_LENGTH, _DIM))
    for pos in range(_LENGTH):
        for i in range(_DIM):
            pe[pos, i] = pos / np.power(10000, (i - i % 2) / _DIM)
    pe[:, 0::2] = np.sin(pe[:, 0::2])
    pe[:, 1::2] = np.cos(pe[:, 1::2])
    return pe.astype(np.float32)


def _pe_tail():
    return jnp.asarray(_pe_full()[:, 4:])


def _pe_head():
    pe = _pe_full()
    tab = np.zeros((_LENGTH, 16), np.float32)
    for m in range(_LENGTH):
        for k in range(4):
            tab[m, 4 * k : 4 * k + 4] = pe[(m + k) % _LENGTH, :4]
    return jnp.asarray(tab)


def _sc_lookup(idx, table, pe_tail, pe_head):
    mesh = plsc.VectorSubcoreMesh(core_axis_name="c", subcore_axis_name="s")

    @functools.partial(
        pl.kernel,
        mesh=mesh,
        out_type=jax.ShapeDtypeStruct((_BATCH, _LENGTH, 2 * _DIM), jnp.float32),
        scratch_types=[
            pltpu.VMEM((_SPW * _LENGTH,), jnp.int32),
            pltpu.VMEM((_NBUF, _GROWS, _PAD), jnp.float32),
            pltpu.VMEM((_LENGTH, _TAIL), jnp.float32),
            pltpu.VMEM((_LENGTH, 16), jnp.float32),
            pltpu.SemaphoreType.DMA((_NBUF,)),
            pltpu.SemaphoreType.DMA((_NBUF,)),
            pltpu.SemaphoreType.DMA,
        ],
        compiler_params=pltpu.CompilerParams(
            use_tc_tiling_on_sc=False, needs_layout_passes=False
        ),
    )
    def body(idx_hbm, table_hbm, pet_hbm, peh_hbm, out_hbm,
             idx_v, gbuf, petv, pehv, sg, sw, sp):
        wid = lax.axis_index("s") * 2 + lax.axis_index("c")
        s_base = wid * _SPW
        pltpu.sync_copy(idx_hbm.at[pl.ds(s_base * _LENGTH, _SPW * _LENGTH)], idx_v)
        pltpu.sync_copy(pet_hbm, petv)
        pltpu.sync_copy(peh_hbm, pehv)

        lanes = lax.iota(jnp.int32, 16)
        row_off = lanes >> 2
        col_idx = _DIM + (lanes & 3)

        def gather_into(g, b):
            pltpu.async_copy(
                table_hbm.at[idx_v.at[pl.ds(g * _GROWS, _GROWS)]],
                gbuf.at[b], sg.at[b],
            )

        def emb_write(g, b, k):
            return pltpu.make_async_copy(
                gbuf.at[b, pl.ds(k * _LENGTH, _LENGTH), :],
                out_hbm.at[s_base + g * _GS + k, :, pl.ds(0, _PAD)],
                sw.at[b],
            )

        for k in range(_NBUF - 1):
            gather_into(k, k)

        def step(g, carry):
            b = lax.rem(g, _NBUF)
            bn = lax.rem(g + _NBUF - 1, _NBUF)

            @pl.when(g + _NBUF - 1 < _NG)
            def _fire():
                @pl.when(g >= 1)
                def _drain():
                    for k in range(_GS):
                        emb_write(g - 1, bn, k).wait()

                gather_into(g + _NBUF - 1, bn)

            for k in range(_GS):
                pltpu.async_copy(
                    petv,
                    out_hbm.at[s_base + g * _GS + k, :, pl.ds(_PAD, _TAIL)],
                    sp,
                )

            pltpu.make_async_copy(
                table_hbm.at[idx_v.at[pl.ds(g * _GROWS, _GROWS)]],
                gbuf.at[b], sg.at[b],
            ).wait()

            for q in range(_GROWS // 4):
                plsc.store_scatter(
                    gbuf.at[b], [4 * q + row_off, col_idx],
                    pehv[(4 * q) % _LENGTH],
                )

            for k in range(_GS):
                pltpu.async_copy(
                    gbuf.at[b, pl.ds(k * _LENGTH, _LENGTH), :],
                    out_hbm.at[s_base + g * _GS + k, :, pl.ds(0, _PAD)],
                    sw.at[b],
                )
            return carry

        lax.fori_loop(0, _NG, step, 0)

        for g in range(_NG - _NBUF, _NG):
            for k in range(_GS):
                emb_write(jnp.int32(g), g % _NBUF, k).wait()

        def pdrain(s, carry):
            pltpu.make_async_copy(
                petv,
                out_hbm.at[s_base + s, :, pl.ds(_PAD, _TAIL)],
                sp,
            ).wait()
            return carry

        lax.fori_loop(0, _SPW, pdrain, 0)

    return body(idx, table, pe_tail, pe_head)


def kernel(inputs, embeddings):
    idx = inputs.reshape(-1)
    tpad = jnp.pad(embeddings, ((0, 0), (0, _PAD - _DIM)))
    return _sc_lookup(idx, tpad, _pe_tail(), _pe_head())


# speedup vs baseline: 4.0348x; 1.5470x over previous
import functools

import numpy as np
import jax
import jax.numpy as jnp
from jax import lax
from jax.experimental import pallas as pl
from jax.experimental.pallas import tpu as pltpu
from jax.experimental.pallas import tpu_sc as plsc

_LENGTH = 100
_DIM = 100
_PAD = 104
_BATCH = 4096
_NW = 32
_BPW = _BATCH // _NW
_ETILES = _PAD // 8
_PTILES = 25 - _ETILES
_NGB = 3
_NTB = 2


def _pe_full():
    pe = np.zeros((_LENGTH, _DIM))
    for pos in range(_LENGTH):
        for i in range(_DIM):
            pe[pos, i] = pos / np.power(10000, (i - i % 2) / _DIM)
    pe[:, 0::2] = np.sin(pe[:, 0::2])
    pe[:, 1::2] = np.cos(pe[:, 1::2])
    return pe.astype(np.float32)


def _pe_tiles():
    pe = _pe_full()
    t = pe[:, 4:]
    t = t.reshape(_LENGTH, _PTILES, 8, 1)
    return jnp.asarray(np.broadcast_to(t, (_LENGTH, _PTILES, 8, 128)).copy())


def _pe_head():
    pe = _pe_full()
    h = pe[:, :4].reshape(_LENGTH, 4, 1)
    return jnp.asarray(np.broadcast_to(h, (_LENGTH, 4, 16)).copy())


def _sc_lookup(idx_t, table, pe_tiles, pe_head):
    mesh = plsc.VectorSubcoreMesh(core_axis_name="c", subcore_axis_name="s")

    @functools.partial(
        pl.kernel,
        mesh=mesh,
        out_type=jax.ShapeDtypeStruct((_LENGTH, 25, _NW, 8, 128), jnp.float32),
        scratch_types=[
            pltpu.VMEM((_LENGTH, _BPW), jnp.int32),
            pltpu.VMEM((_NGB, _BPW, _PAD), jnp.float32),
            pltpu.VMEM((_NTB, _ETILES, 8, 128), jnp.float32),
            pltpu.VMEM((_LENGTH, 4, 16), jnp.float32),
            pltpu.VMEM_SHARED((_LENGTH // 2, _PTILES, 8, 128), jnp.float32),
            pltpu.SemaphoreType.DMA((_NGB,)),
            pltpu.SemaphoreType.DMA((_NTB,)),
            pltpu.SemaphoreType.DMA,
        ],
        compiler_params=pltpu.CompilerParams(
            use_tc_tiling_on_sc=False, needs_layout_passes=False
        ),
    )
    def body(idx_hbm, table_hbm, pet_hbm, peh_hbm, out_hbm,
             idx_v, gbuf, tbuf, pehv, shpe, sg, sw, sp):
        cid = lax.axis_index("c")
        sid = lax.axis_index("s")
        wid = sid * 2 + cid

        for k in range(4):
            ll = sid + 16 * k

            @pl.when(ll < _LENGTH // 2)
            def _load():
                pltpu.sync_copy(pet_hbm.at[cid * (_LENGTH // 2) + ll], shpe.at[ll])

        pltpu.sync_copy(idx_hbm.at[:, pl.ds(wid * _BPW, _BPW)], idx_v)
        pltpu.sync_copy(peh_hbm, pehv)
        plsc.subcore_barrier()

        lanes = lax.iota(jnp.int32, 16)
        rowv = [b0 * 16 + lanes for b0 in range(8)]

        def gather_into(l, b):
            pltpu.async_copy(table_hbm.at[idx_v.at[l]], gbuf.at[b], sg.at[b])

        def twrite(l, tb):
            return pltpu.make_async_copy(
                tbuf.at[tb],
                out_hbm.at[l, pl.ds(0, _ETILES), wid, :, :],
                sw.at[tb],
            )

        for k in range(_NGB):
            gather_into(k, k)

        def step(l, carry):
            b = lax.rem(l, _NGB)
            tb = lax.rem(l, _NTB)

            pltpu.make_async_copy(
                table_hbm.at[idx_v.at[l]], gbuf.at[b], sg.at[b]
            ).wait()

            @pl.when(l >= _NTB)
            def _drain():
                twrite(l - _NTB, tb).wait()

            def tcol(t2, carry2):
                for s2 in range(8):
                    colv = jnp.broadcast_to(t2 * 8 + s2, (16,)).astype(jnp.int32)
                    for b0 in range(8):
                        v = plsc.load_gather(gbuf.at[b], [rowv[b0], colv])
                        tbuf[tb, t2, s2, pl.ds(b0 * 16, 16)] = v
                return carry2

            lax.fori_loop(0, _ETILES - 1, tcol, 0)

            for s2 in range(4):
                colv = jnp.broadcast_to(96 + s2, (16,)).astype(jnp.int32)
                for b0 in range(8):
                    v = plsc.load_gather(gbuf.at[b], [rowv[b0], colv])
                    tbuf[tb, _ETILES - 1, s2, pl.ds(b0 * 16, 16)] = v
            for s2 in range(4):
                v = pehv[l, s2, :]
                for b0 in range(8):
                    tbuf[tb, _ETILES - 1, 4 + s2, pl.ds(b0 * 16, 16)] = v

            twrite(l, tb).start()
            flat = sid * _LENGTH + l
            ll = flat // 32
            t0p = lax.rem(flat, 32)
            pltpu.async_copy(
                shpe.at[ll],
                out_hbm.at[cid * (_LENGTH // 2) + ll,
                           pl.ds(_ETILES, _PTILES), t0p, :, :],
                sp,
            )

            @pl.when(l + _NGB < _LENGTH)
            def _fire():
                gather_into(l + _NGB, b)

            return carry

        lax.fori_loop(0, _LENGTH, step, 0)

        for k in range(_NTB):
            l = _LENGTH - _NTB + k
            twrite(l, l % _NTB).wait()

        def pdrain(l, carry):
            flat = sid * _LENGTH + l
            ll = flat // 32
            t0p = lax.rem(flat, 32)
            pltpu.make_async_copy(
                shpe.at[ll],
                out_hbm.at[cid * (_LENGTH // 2) + ll,
                           pl.ds(_ETILES, _PTILES), t0p, :, :],
                sp,
            ).wait()
            return carry

        lax.fori_loop(0, _LENGTH, pdrain, 0)

    return body(idx_t, table, pe_tiles, pe_head)


def kernel(inputs, embeddings):
    idx_t = inputs.T
    tpad = jnp.pad(embeddings, ((0, 0), (0, _PAD - _DIM)))
    out5 = _sc_lookup(idx_t, tpad, _pe_tiles(), _pe_head())
    t = jnp.transpose(out5, (2, 4, 0, 1, 3))
    return t.reshape(_BATCH, _LENGTH, 2 * _DIM)


# speedup vs baseline: 4.9597x; 1.2292x over previous
import functools

import numpy as np
import jax
import jax.numpy as jnp
from jax import lax
from jax.experimental import pallas as pl
from jax.experimental.pallas import tpu as pltpu
from jax.experimental.pallas import tpu_sc as plsc

_LENGTH = 100
_DIM = 100
_PAD = 104
_BATCH = 4096
_NW = 32
_BPW = _BATCH // _NW
_ETILES = _PAD // 8
_PTILES = 25 - _ETILES
_NGB = 3
_NTB = 2


def _pe_full():
    pe = np.zeros((_LENGTH, _DIM))
    for pos in range(_LENGTH):
        for i in range(_DIM):
            pe[pos, i] = pos / np.power(10000, (i - i % 2) / _DIM)
    pe[:, 0::2] = np.sin(pe[:, 0::2])
    pe[:, 1::2] = np.cos(pe[:, 1::2])
    return pe.astype(np.float32)


def _pe_tiles():
    pe = _pe_full()
    t = pe[:, 4:]
    t = t.reshape(_LENGTH, _PTILES, 8, 1)
    return jnp.asarray(np.broadcast_to(t, (_LENGTH, _PTILES, 8, 128)).copy())


def _pe_head():
    pe = _pe_full()
    h = pe[:, :4].reshape(_LENGTH, 4, 1)
    return jnp.asarray(np.broadcast_to(h, (_LENGTH, 4, 16)).copy())


def _sc_lookup(idx_t, table, pe_tiles, pe_head):
    mesh = plsc.VectorSubcoreMesh(core_axis_name="c", subcore_axis_name="s")

    @functools.partial(
        pl.kernel,
        mesh=mesh,
        out_type=jax.ShapeDtypeStruct((_LENGTH, 25, _NW, 8, 128), jnp.float32),
        scratch_types=[
            pltpu.VMEM((_LENGTH, _BPW), jnp.int32),
            pltpu.VMEM((_NGB, _BPW, _PAD), jnp.float32),
            pltpu.VMEM((_NTB, _ETILES, 8, 128), jnp.float32),
            pltpu.VMEM((_LENGTH, 4, 16), jnp.float32),
            pltpu.VMEM_SHARED((_LENGTH // 2, _PTILES, 8, 128), jnp.float32),
            pltpu.SemaphoreType.DMA((_NGB,)),
            pltpu.SemaphoreType.DMA((_NTB,)),
            pltpu.SemaphoreType.DMA,
        ],
        compiler_params=pltpu.CompilerParams(
            use_tc_tiling_on_sc=False, needs_layout_passes=False
        ),
    )
    def body(idx_hbm, table_hbm, pet_hbm, peh_hbm, out_hbm,
             idx_v, gbuf, tbuf, pehv, shpe, sg, sw, sp):
        cid = lax.axis_index("c")
        sid = lax.axis_index("s")
        wid = sid * 2 + cid

        for k in range(4):
            ll = sid + 16 * k

            @pl.when(ll < _LENGTH // 2)
            def _load():
                pltpu.sync_copy(pet_hbm.at[cid * (_LENGTH // 2) + ll], shpe.at[ll])

        pltpu.sync_copy(idx_hbm.at[:, pl.ds(wid * _BPW, _BPW)], idx_v)
        pltpu.sync_copy(peh_hbm, pehv)
        plsc.subcore_barrier()

        lanes = lax.iota(jnp.int32, 16)
        rowv = [b0 * 16 + lanes for b0 in range(8)]

        def gather_into(l, b):
            pltpu.async_copy(table_hbm.at[idx_v.at[l]], gbuf.at[b], sg.at[b])

        def twrite(l, tb):
            return pltpu.make_async_copy(
                tbuf.at[tb],
                out_hbm.at[l, pl.ds(0, _ETILES), wid, :, :],
                sw.at[tb],
            )

        for k in range(_NGB):
            gather_into(k, k)

        def step(l, carry):
            b = lax.rem(l, _NGB)
            tb = lax.rem(l, _NTB)

            pltpu.make_async_copy(
                table_hbm.at[idx_v.at[l]], gbuf.at[b], sg.at[b]
            ).wait()

            @pl.when(l >= _NTB)
            def _drain():
                twrite(l - _NTB, tb).wait()

            for c in range(_DIM):
                colv = jnp.full((16,), c, jnp.int32)
                vs = [
                    plsc.load_gather(gbuf.at[b], [rowv[b0], colv])
                    for b0 in range(8)
                ]
                for b0 in range(8):
                    tbuf[tb, c // 8, c % 8, pl.ds(b0 * 16, 16)] = vs[b0]

            for s2 in range(4):
                v = pehv[l, s2, :]
                for b0 in range(8):
                    tbuf[tb, _ETILES - 1, 4 + s2, pl.ds(b0 * 16, 16)] = v

            twrite(l, tb).start()
            flat = sid * _LENGTH + l
            ll = flat // 32
            t0p = lax.rem(flat, 32)
            pltpu.async_copy(
                shpe.at[ll],
                out_hbm.at[cid * (_LENGTH // 2) + ll,
                           pl.ds(_ETILES, _PTILES), t0p, :, :],
                sp,
            )

            @pl.when(l + _NGB < _LENGTH)
            def _fire():
                gather_into(l + _NGB, b)

            return carry

        lax.fori_loop(0, _LENGTH, step, 0)

        for k in range(_NTB):
            l = _LENGTH - _NTB + k
            twrite(l, l % _NTB).wait()

        def pdrain(l, carry):
            flat = sid * _LENGTH + l
            ll = flat // 32
            t0p = lax.rem(flat, 32)
            pltpu.make_async_copy(
                shpe.at[ll],
                out_hbm.at[cid * (_LENGTH // 2) + ll,
                           pl.ds(_ETILES, _PTILES), t0p, :, :],
                sp,
            ).wait()
            return carry

        lax.fori_loop(0, _LENGTH, pdrain, 0)

    return body(idx_t, table, pe_tiles, pe_head)


def kernel(inputs, embeddings):
    idx_t = inputs.T
    tpad = jnp.pad(embeddings, ((0, 0), (0, _PAD - _DIM)))
    out5 = _sc_lookup(idx_t, tpad, _pe_tiles(), _pe_head())
    t = jnp.transpose(out5, (2, 4, 0, 1, 3))
    return t.reshape(_BATCH, _LENGTH, 2 * _DIM)


# speedup vs baseline: 5.2187x; 1.0522x over previous
import functools

import numpy as np
import jax
import jax.numpy as jnp
from jax import lax
from jax.experimental import pallas as pl
from jax.experimental.pallas import tpu as pltpu
from jax.experimental.pallas import tpu_sc as plsc

_LENGTH = 100
_DIM = 100
_PAD = 104
_BATCH = 4096
_NW = 32
_BPW = _BATCH // _NW
_ETILES = _PAD // 8
_PTILES = 25 - _ETILES
_NGB = 3
_NTB = 2


def _pe_full():
    pe = np.zeros((_LENGTH, _DIM))
    for pos in range(_LENGTH):
        for i in range(_DIM):
            pe[pos, i] = pos / np.power(10000, (i - i % 2) / _DIM)
    pe[:, 0::2] = np.sin(pe[:, 0::2])
    pe[:, 1::2] = np.cos(pe[:, 1::2])
    return pe.astype(np.float32)


def _pe_tiles():
    pe = _pe_full()
    t = pe[:, 4:]
    t = t.reshape(_LENGTH, _PTILES, 8, 1)
    return jnp.asarray(np.broadcast_to(t, (_LENGTH, _PTILES, 8, 128)).copy())


def _pe_head():
    pe = _pe_full()
    h = pe[:, :4].reshape(_LENGTH, 4, 1)
    return jnp.asarray(np.broadcast_to(h, (_LENGTH, 4, 16)).copy())


def _sc_lookup(idx_t, table, pe_tiles, pe_head):
    mesh = plsc.VectorSubcoreMesh(core_axis_name="c", subcore_axis_name="s")

    @functools.partial(
        pl.kernel,
        mesh=mesh,
        out_type=jax.ShapeDtypeStruct((_LENGTH, 25, _NW, 8, 128), jnp.float32),
        scratch_types=[
            pltpu.VMEM((_LENGTH, _BPW), jnp.int32),
            pltpu.VMEM((_NGB, _BPW, _PAD), jnp.float32),
            pltpu.VMEM((_NTB, _ETILES, 8, 128), jnp.float32),
            pltpu.VMEM((_LENGTH, 4, 16), jnp.float32),
            pltpu.VMEM_SHARED((_LENGTH // 2, _PTILES, 8, 128), jnp.float32),
            pltpu.SemaphoreType.DMA((_NGB,)),
            pltpu.SemaphoreType.DMA((_NTB,)),
            pltpu.SemaphoreType.DMA,
        ],
        compiler_params=pltpu.CompilerParams(
            use_tc_tiling_on_sc=False, needs_layout_passes=False
        ),
    )
    def body(idx_hbm, table_hbm, pet_hbm, peh_hbm, out_hbm,
             idx_v, gbuf, tbuf, pehv, shpe, sg, sw, sp):
        cid = lax.axis_index("c")
        sid = lax.axis_index("s")
        wid = sid * 2 + cid

        for k in range(4):
            ll = sid + 16 * k

            @pl.when(ll < _LENGTH // 2)
            def _load():
                pltpu.sync_copy(pet_hbm.at[cid * (_LENGTH // 2) + ll], shpe.at[ll])

        pltpu.sync_copy(idx_hbm.at[:, pl.ds(wid * _BPW, _BPW)], idx_v)
        pltpu.sync_copy(peh_hbm, pehv)
        plsc.subcore_barrier()

        lanes = lax.iota(jnp.int32, 16)
        rowv = [b0 * 16 + lanes for b0 in range(8)]

        def gather_into(l, b):
            pltpu.async_copy(table_hbm.at[idx_v.at[l]], gbuf.at[b], sg.at[b])

        def twrite(l, tb):
            return pltpu.make_async_copy(
                tbuf.at[tb],
                out_hbm.at[l, pl.ds(0, _ETILES), wid, :, :],
                sw.at[tb],
            )

        for k in range(_NGB):
            gather_into(k, k)

        def step(l, carry):
            b = lax.rem(l, _NGB)
            tb = lax.rem(l, _NTB)

            pltpu.make_async_copy(
                table_hbm.at[idx_v.at[l]], gbuf.at[b], sg.at[b]
            ).wait()

            @pl.when(l >= _NTB)
            def _drain():
                twrite(l - _NTB, tb).wait()

            for c0 in range(0, _DIM, 2):
                vs = []
                for c in (c0, c0 + 1):
                    colv = jnp.full((16,), c, jnp.int32)
                    vs += [
                        plsc.load_gather(gbuf.at[b], [rowv[b0], colv])
                        for b0 in range(8)
                    ]
                for k, c in enumerate((c0, c0 + 1)):
                    for b0 in range(8):
                        tbuf[tb, c // 8, c % 8, pl.ds(b0 * 16, 16)] = vs[8 * k + b0]

            for s2 in range(4):
                v = pehv[l, s2, :]
                for b0 in range(8):
                    tbuf[tb, _ETILES - 1, 4 + s2, pl.ds(b0 * 16, 16)] = v

            twrite(l, tb).start()
            flat = sid * _LENGTH + l
            ll = flat // 32
            t0p = lax.rem(flat, 32)
            pltpu.async_copy(
                shpe.at[ll],
                out_hbm.at[cid * (_LENGTH // 2) + ll,
                           pl.ds(_ETILES, _PTILES), t0p, :, :],
                sp,
            )

            @pl.when(l + _NGB < _LENGTH)
            def _fire():
                gather_into(l + _NGB, b)

            return carry

        lax.fori_loop(0, _LENGTH, step, 0)

        for k in range(_NTB):
            l = _LENGTH - _NTB + k
            twrite(l, l % _NTB).wait()

        def pdrain(l, carry):
            flat = sid * _LENGTH + l
            ll = flat // 32
            t0p = lax.rem(flat, 32)
            pltpu.make_async_copy(
                shpe.at[ll],
                out_hbm.at[cid * (_LENGTH // 2) + ll,
                           pl.ds(_ETILES, _PTILES), t0p, :, :],
                sp,
            ).wait()
            return carry

        lax.fori_loop(0, _LENGTH, pdrain, 0)

    return body(idx_t, table, pe_tiles, pe_head)


def kernel(inputs, embeddings):
    idx_t = inputs.T
    tpad = jnp.pad(embeddings, ((0, 0), (0, _PAD - _DIM)))
    out5 = _sc_lookup(idx_t, tpad, _pe_tiles(), _pe_head())
    t = jnp.transpose(out5, (2, 4, 0, 1, 3))
    return t.reshape(_BATCH, _LENGTH, 2 * _DIM)
